# baseline (device time: 13611 ns/iter reference)
import jax
import jax.numpy as jnp
from jax import lax
from jax.experimental import pallas as pl
from jax.experimental.pallas import tpu as pltpu

T = 256
V_LOCAL = 4096


def kernel(x, W, labels):
    def body(x_ref, w_ref, labels_ref, out_ref,
             stats_ref, recv_ref, send_sem, recv_sem):
        my_x = lax.axis_index("x")
        my_y = lax.axis_index("y")
        nbr = (my_x, 1 - my_y)

        barrier_sem = pltpu.get_barrier_semaphore()
        pl.semaphore_signal(barrier_sem, inc=1, device_id=nbr,
                            device_id_type=pl.DeviceIdType.MESH)
        pl.semaphore_wait(barrier_sem, 1)

        xb = x_ref[:].astype(jnp.bfloat16)
        wb = w_ref[:].astype(jnp.bfloat16)
        logits = jnp.dot(xb, wb, preferred_element_type=jnp.float32)

        m = jnp.max(logits, axis=1, keepdims=True)
        s = jnp.sum(jnp.exp(logits - m), axis=1, keepdims=True)

        idx = labels_ref[:] - my_y * V_LOCAL
        cols = lax.broadcasted_iota(jnp.int32, (T, V_LOCAL), 1)
        ll = jnp.sum(jnp.where(cols == idx, logits, 0.0),
                     axis=1, keepdims=True)

        stats_ref[:, 0:1] = m
        stats_ref[:, 1:2] = s
        stats_ref[:, 2:3] = ll
        stats_ref[:, 3:4] = jnp.zeros_like(m)

        rdma = pltpu.make_async_remote_copy(
            src_ref=stats_ref,
            dst_ref=recv_ref,
            send_sem=send_sem,
            recv_sem=recv_sem,
            device_id=nbr,
            device_id_type=pl.DeviceIdType.MESH,
        )
        rdma.start()
        rdma.wait()

        mr = recv_ref[:, 0:1]
        sr = recv_ref[:, 1:2]
        llr = recv_ref[:, 2:3]
        m2 = jnp.maximum(m, mr)
        s2 = jnp.exp(m - m2) * s + jnp.exp(mr - m2) * sr
        out_ref[:] = m2 + jnp.log(s2) - (ll + llr)

    out = pl.pallas_call(
        body,
        out_shape=jax.ShapeDtypeStruct((T, 1), jnp.float32),
        in_specs=[
            pl.BlockSpec(memory_space=pltpu.VMEM),
            pl.BlockSpec(memory_space=pltpu.VMEM),
            pl.BlockSpec(memory_space=pltpu.VMEM),
        ],
        out_specs=pl.BlockSpec(memory_space=pltpu.VMEM),
        scratch_shapes=[
            pltpu.VMEM((T, 4), jnp.float32),
            pltpu.VMEM((T, 4), jnp.float32),
            pltpu.SemaphoreType.DMA,
            pltpu.SemaphoreType.DMA,
        ],
        compiler_params=pltpu.CompilerParams(collective_id=0),
    )(x, W, labels.reshape(T, 1))
    return out.reshape(T)


# device time: 12798 ns/iter; 1.0635x vs baseline; 1.0635x over previous
import jax
import jax.numpy as jnp
from jax import lax
from jax.experimental import pallas as pl
from jax.experimental.pallas import tpu as pltpu

T = 256
V_LOCAL = 4096


def kernel(x, W, labels):
    def body(x_ref, w_ref, labels_ref, out_ref,
             stats_ref, recv_ref, send_sem, recv_sem):
        my_x = lax.axis_index("x")
        my_y = lax.axis_index("y")
        nbr = (my_x, 1 - my_y)

        barrier_sem = pltpu.get_barrier_semaphore()
        pl.semaphore_signal(barrier_sem, inc=1, device_id=nbr,
                            device_id_type=pl.DeviceIdType.MESH)
        pl.semaphore_wait(barrier_sem, 1)

        xb = x_ref[:].astype(jnp.bfloat16)
        idx = labels_ref[:] - my_y * V_LOCAL

        CHUNK = 512
        s = jnp.zeros((T, 1), jnp.float32)
        ll = jnp.zeros((T, 1), jnp.float32)
        for c in range(V_LOCAL // CHUNK):
            wb = w_ref[:, c * CHUNK:(c + 1) * CHUNK].astype(jnp.bfloat16)
            lg = jnp.dot(xb, wb, preferred_element_type=jnp.float32)
            s = s + jnp.sum(jnp.exp(lg), axis=1, keepdims=True)
            cols = c * CHUNK + lax.broadcasted_iota(jnp.int32, (T, CHUNK), 1)
            ll = ll + jnp.sum(jnp.where(cols == idx, lg, 0.0),
                              axis=1, keepdims=True)

        stats_ref[:, 0:1] = s
        stats_ref[:, 1:2] = ll
        stats_ref[:, 2:4] = jnp.zeros((T, 2), jnp.float32)

        rdma = pltpu.make_async_remote_copy(
            src_ref=stats_ref,
            dst_ref=recv_ref,
            send_sem=send_sem,
            recv_sem=recv_sem,
            device_id=nbr,
            device_id_type=pl.DeviceIdType.MESH,
        )
        rdma.start()
        rdma.wait()

        sr = recv_ref[:, 0:1]
        llr = recv_ref[:, 1:2]
        out_ref[:] = jnp.log(s + sr) - (ll + llr)

    out = pl.pallas_call(
        body,
        out_shape=jax.ShapeDtypeStruct((T, 1), jnp.float32),
        in_specs=[
            pl.BlockSpec(memory_space=pltpu.VMEM),
            pl.BlockSpec(memory_space=pltpu.VMEM),
            pl.BlockSpec(memory_space=pltpu.VMEM),
        ],
        out_specs=pl.BlockSpec(memory_space=pltpu.VMEM),
        scratch_shapes=[
            pltpu.VMEM((T, 4), jnp.float32),
            pltpu.VMEM((T, 4), jnp.float32),
            pltpu.SemaphoreType.DMA,
            pltpu.SemaphoreType.DMA,
        ],
        compiler_params=pltpu.CompilerParams(collective_id=0),
    )(x, W, labels.reshape(T, 1))
    return out.reshape(T)


# device time: 11406 ns/iter; 1.1933x vs baseline; 1.1220x over previous
import jax
import jax.numpy as jnp
from jax import lax
from jax.experimental import pallas as pl
from jax.experimental.pallas import tpu as pltpu

T = 256
D = 512
V_LOCAL = 4096
CHUNK = 512
N_CHUNKS = V_LOCAL // CHUNK


def kernel(x, W, labels):
    def body(x_ref, w_ref, labels_ref, out_ref,
             xb_ref, acc_s, acc_ll, stats_ref, recv_ref, send_sem, recv_sem):
        c = pl.program_id(0)
        my_x = lax.axis_index("x")
        my_y = lax.axis_index("y")
        nbr = (my_x, 1 - my_y)

        @pl.when(c == 0)
        def _():
            barrier_sem = pltpu.get_barrier_semaphore()
            pl.semaphore_signal(barrier_sem, inc=1, device_id=nbr,
                                device_id_type=pl.DeviceIdType.MESH)
            pl.semaphore_wait(barrier_sem, 1)
            xb_ref[:, :] = x_ref[:, :].astype(jnp.bfloat16)
            acc_s[:, :] = jnp.zeros((1, T), jnp.float32)
            acc_ll[:, :] = jnp.zeros((1, T), jnp.float32)

        wb = w_ref[:, :].astype(jnp.bfloat16)
        lgT = lax.dot_general(wb, xb_ref[:, :], (((0,), (1,)), ((), ())),
                              preferred_element_type=jnp.float32)
        acc_s[:, :] += jnp.sum(jnp.exp(lgT), axis=0, keepdims=True)
        rows = c * CHUNK + lax.broadcasted_iota(jnp.int32, (CHUNK, T), 0)
        idx = labels_ref[:, :] - my_y * V_LOCAL
        acc_ll[:, :] += jnp.sum(jnp.where(rows == idx, lgT, 0.0),
                                axis=0, keepdims=True)

        @pl.when(c == N_CHUNKS - 1)
        def _():
            stats_ref[0:1, :] = acc_s[:, :]
            stats_ref[1:2, :] = acc_ll[:, :]
            rdma = pltpu.make_async_remote_copy(
                src_ref=stats_ref, dst_ref=recv_ref,
                send_sem=send_sem, recv_sem=recv_sem,
                device_id=nbr, device_id_type=pl.DeviceIdType.MESH,
            )
            rdma.start()
            rdma.wait()
            out_ref[:, :] = (jnp.log(acc_s[:, :] + recv_ref[0:1, :])
                             - (acc_ll[:, :] + recv_ref[1:2, :]))

    out = pl.pallas_call(
        body,
        grid=(N_CHUNKS,),
        out_shape=jax.ShapeDtypeStruct((1, T), jnp.float32),
        in_specs=[
            pl.BlockSpec((T, D), lambda c: (0, 0)),
            pl.BlockSpec((D, CHUNK), lambda c: (0, c)),
            pl.BlockSpec((1, T), lambda c: (0, 0)),
        ],
        out_specs=pl.BlockSpec((1, T), lambda c: (0, 0)),
        scratch_shapes=[
            pltpu.VMEM((T, D), jnp.bfloat16),
            pltpu.VMEM((1, T), jnp.float32),
            pltpu.VMEM((1, T), jnp.float32),
            pltpu.VMEM((2, T), jnp.float32),
            pltpu.VMEM((2, T), jnp.float32),
            pltpu.SemaphoreType.DMA,
            pltpu.SemaphoreType.DMA,
        ],
        compiler_params=pltpu.CompilerParams(collective_id=0),
    )(x, W, labels.reshape(1, T))
    return out.reshape(T)
